# baseline (speedup 1.0000x reference)
import functools

import jax
import jax.numpy as jnp
from jax import lax
from jax.experimental import pallas as pl
from jax.experimental.pallas import tpu as pltpu
from jax.experimental.pallas import tpu_sc as plsc

VOCAB = 1000000
DIM = 64
BATCH = 4096
SEQ = 50
PAD_IDX = 0

L = 16
NW = 32
B_PER_W = BATCH // NW


def _encoder_kernel(tokens_hbm, table_hbm, out_hbm,
                    idx_v, rows0_v, rows1_v, row0_v, out_v,
                    sem0, sem1):
    cid = lax.axis_index("c")
    sid = lax.axis_index("s")
    wid = sid * 2 + cid

    pltpu.sync_copy(tokens_hbm.at[pl.ds(wid * B_PER_W, B_PER_W)], idx_v)
    pltpu.sync_copy(table_hbm.at[pl.ds(0, 1)], row0_v)

    iota = lax.iota(jnp.int32, L)
    one = jnp.ones((L,), jnp.int32)
    izero = jnp.zeros((L,), jnp.int32)
    zeros = jnp.zeros((L,), jnp.float32)
    row0 = [row0_v[0, pl.ds(k * L, L)] for k in range(4)]

    def count_nonpad(row):
        cnt = izero
        for off in (0, L, 2 * L):
            toks = idx_v[row, pl.ds(off, L)]
            cnt = cnt + jnp.where(toks != PAD_IDX, one, izero)
        tail = idx_v[row, pl.ds(34, L)]
        cnt = cnt + jnp.where(
            jnp.logical_and(iota >= 14, tail != PAD_IDX), one, izero)
        parts = [cnt[i] for i in range(L)]
        while len(parts) > 1:
            parts = [parts[i] + parts[i + 1] for i in range(0, len(parts), 2)]
        return parts[0]

    def compute_row(row, rows_v):
        n1 = jnp.full((L,), count_nonpad(row), jnp.float32)
        recip = 1.0 / jnp.maximum(n1, 1.0)
        n0f = (SEQ - n1) * recip
        acc = [zeros, zeros, zeros, zeros]
        for t in range(SEQ):
            for k in range(4):
                acc[k] = acc[k] + rows_v[t, pl.ds(k * L, L)]
        for k in range(4):
            out_v[row, pl.ds(k * L, L)] = acc[k] * recip - n0f * row0[k]

    def gather(row, rows_v, sem):
        return pltpu.async_copy(table_hbm.at[idx_v.at[row]], rows_v, sem)

    gather(0, rows0_v, sem0).wait()

    def body(i, carry):
        r0 = 2 * i
        gather(r0 + 1, rows1_v, sem1)
        compute_row(r0, rows0_v)
        pltpu.make_async_copy(table_hbm.at[idx_v.at[r0 + 1]], rows1_v,
                              sem1).wait()

        @pl.when(r0 + 2 < B_PER_W)
        def _():
            gather(r0 + 2, rows0_v, sem0)
        compute_row(r0 + 1, rows1_v)

        @pl.when(r0 + 2 < B_PER_W)
        def _():
            pltpu.make_async_copy(table_hbm.at[idx_v.at[r0 + 2]], rows0_v,
                                  sem0).wait()
        return carry

    lax.fori_loop(0, B_PER_W // 2, body, 0)

    pltpu.sync_copy(out_v, out_hbm.at[pl.ds(wid * B_PER_W, B_PER_W)])


@jax.jit
def kernel(token_ids, table):
    mesh = plsc.VectorSubcoreMesh(core_axis_name="c", subcore_axis_name="s")
    f = functools.partial(
        pl.kernel,
        mesh=mesh,
        compiler_params=pltpu.CompilerParams(use_tc_tiling_on_sc=False),
        out_type=jax.ShapeDtypeStruct((BATCH, DIM), jnp.float32),
        scratch_types=[
            pltpu.VMEM((B_PER_W, SEQ), jnp.int32),
            pltpu.VMEM((SEQ, DIM), jnp.float32),
            pltpu.VMEM((SEQ, DIM), jnp.float32),
            pltpu.VMEM((1, DIM), jnp.float32),
            pltpu.VMEM((B_PER_W, DIM), jnp.float32),
            pltpu.SemaphoreType.DMA,
            pltpu.SemaphoreType.DMA,
        ],
    )(_encoder_kernel)
    return f(token_ids, table)


# speedup vs baseline: 1.0501x; 1.0501x over previous
import functools

import jax
import jax.numpy as jnp
from jax import lax
from jax.experimental import pallas as pl
from jax.experimental.pallas import tpu as pltpu
from jax.experimental.pallas import tpu_sc as plsc

VOCAB = 1000000
DIM = 64
BATCH = 4096
SEQ = 50
PAD_IDX = 0

L = 16
NW = 32
B_PER_W = BATCH // NW
SEQ_PAD = 128
ROWS_PER_CHUNK = 2
CHUNK_IDX = ROWS_PER_CHUNK * SEQ
N_CHUNKS = B_PER_W // ROWS_PER_CHUNK


def _encoder_kernel(tokens_hbm, table_hbm, out_hbm,
                    idxp_v, idx_v, rows0_v, rows1_v, row0_v, out_v,
                    sem0, sem1):
    cid = lax.axis_index("c")
    sid = lax.axis_index("s")
    wid = sid * 2 + cid

    pltpu.sync_copy(tokens_hbm.at[pl.ds(wid * B_PER_W, B_PER_W)], idxp_v)
    pltpu.sync_copy(table_hbm.at[pl.ds(0, 1)], row0_v)

    for r in range(B_PER_W):
        c, half = r // 2, (r % 2) * SEQ
        for off in (0, L, 2 * L, 34):
            idx_v[c, pl.ds(half + off, L)] = idxp_v[r, pl.ds(off, L)]

    one = jnp.ones((L,), jnp.int32)
    izero = jnp.zeros((L,), jnp.int32)
    zeros = jnp.zeros((L,), jnp.float32)
    row0 = [row0_v[0, pl.ds(k * L, L)] for k in range(4)]

    def count_nonpad(row):
        cnt = izero
        for off in (0, L, 2 * L, 3 * L):
            toks = idxp_v[row, pl.ds(off, L)]
            cnt = cnt + jnp.where(toks != PAD_IDX, one, izero)
        parts = [cnt[i] for i in range(L)]
        while len(parts) > 1:
            parts = [parts[i] + parts[i + 1] for i in range(0, len(parts), 2)]
        return parts[0]

    def compute_chunk(c, rows_v):
        for r in range(ROWS_PER_CHUNK):
            n1 = jnp.full((L,), count_nonpad(2 * c + r), jnp.float32)
            recip = 1.0 / jnp.maximum(n1, 1.0)
            n0f = (SEQ - n1) * recip
            acc = [zeros, zeros, zeros, zeros]
            for t in range(SEQ):
                slot = r * SEQ + t
                for k in range(4):
                    acc[k] = acc[k] + rows_v[slot, pl.ds(k * L, L)]
            orow = 2 * c + r
            for k in range(4):
                out_v[orow, pl.ds(k * L, L)] = acc[k] * recip - n0f * row0[k]

    def gather(c, rows_v, sem):
        return pltpu.async_copy(table_hbm.at[idx_v.at[c]], rows_v, sem)

    gather(0, rows0_v, sem0).wait()

    def body(i, carry):
        c0 = 2 * i
        gather(c0 + 1, rows1_v, sem1)
        compute_chunk(c0, rows0_v)
        pltpu.make_async_copy(table_hbm.at[idx_v.at[c0 + 1]], rows1_v,
                              sem1).wait()

        @pl.when(c0 + 2 < N_CHUNKS)
        def _():
            gather(c0 + 2, rows0_v, sem0)
        compute_chunk(c0 + 1, rows1_v)

        @pl.when(c0 + 2 < N_CHUNKS)
        def _():
            pltpu.make_async_copy(table_hbm.at[idx_v.at[c0 + 2]], rows0_v,
                                  sem0).wait()
        return carry

    lax.fori_loop(0, N_CHUNKS // 2, body, 0)

    pltpu.sync_copy(out_v, out_hbm.at[pl.ds(wid * B_PER_W, B_PER_W)])


@jax.jit
def kernel(token_ids, table):
    tokens_pad = jnp.pad(token_ids, ((0, 0), (0, SEQ_PAD - SEQ)))
    mesh = plsc.VectorSubcoreMesh(core_axis_name="c", subcore_axis_name="s")
    f = functools.partial(
        pl.kernel,
        mesh=mesh,
        compiler_params=pltpu.CompilerParams(use_tc_tiling_on_sc=False),
        out_type=jax.ShapeDtypeStruct((BATCH, DIM), jnp.float32),
        scratch_types=[
            pltpu.VMEM((B_PER_W, SEQ_PAD), jnp.int32),
            pltpu.VMEM((N_CHUNKS, CHUNK_IDX), jnp.int32),
            pltpu.VMEM((CHUNK_IDX, DIM), jnp.float32),
            pltpu.VMEM((CHUNK_IDX, DIM), jnp.float32),
            pltpu.VMEM((1, DIM), jnp.float32),
            pltpu.VMEM((B_PER_W, DIM), jnp.float32),
            pltpu.SemaphoreType.DMA,
            pltpu.SemaphoreType.DMA,
        ],
    )(_encoder_kernel)
    return f(tokens_pad, table)


# speedup vs baseline: 1.1179x; 1.0645x over previous
import functools

import jax
import jax.numpy as jnp
from jax import lax
from jax.experimental import pallas as pl
from jax.experimental.pallas import tpu as pltpu
from jax.experimental.pallas import tpu_sc as plsc

VOCAB = 1000000
DIM = 64
BATCH = 4096
SEQ = 50
PAD_IDX = 0

L = 16
NW = 32
B_PER_W = BATCH // NW
SEQ_PAD = 128
ROWS_PER_CHUNK = 2
CHUNK_IDX = ROWS_PER_CHUNK * SEQ
N_CHUNKS = B_PER_W // ROWS_PER_CHUNK


DIM_PAD = 128


def _encoder_kernel(tokens_hbm, table_hbm, out_hbm,
                    idxp_v, idx_v, rows0_v, rows1_v, row0_v, out_v,
                    sem0, sem1):
    cid = lax.axis_index("c")
    sid = lax.axis_index("s")
    wid = sid * 2 + cid

    pltpu.sync_copy(tokens_hbm.at[pl.ds(wid * B_PER_W, B_PER_W)], idxp_v)
    pltpu.sync_copy(table_hbm.at[pl.ds(0, 1)], row0_v)

    for r in range(B_PER_W):
        c, half = r // 2, (r % 2) * SEQ
        for off in (0, L, 2 * L, 34):
            idx_v[c, pl.ds(half + off, L)] = idxp_v[r, pl.ds(off, L)]

    one = jnp.ones((L,), jnp.int32)
    izero = jnp.zeros((L,), jnp.int32)
    zeros = jnp.zeros((L,), jnp.float32)
    row0 = [row0_v[0, pl.ds(k * L, L)] for k in range(4)]

    def count_nonpad(row):
        cnt = izero
        for off in (0, L, 2 * L, 3 * L):
            toks = idxp_v[row, pl.ds(off, L)]
            cnt = cnt + jnp.where(toks != PAD_IDX, one, izero)
        parts = [cnt[i] for i in range(L)]
        while len(parts) > 1:
            parts = [parts[i] + parts[i + 1] for i in range(0, len(parts), 2)]
        return parts[0]

    def compute_chunk(c, rows_v):
        for r in range(ROWS_PER_CHUNK):
            n1 = jnp.full((L,), count_nonpad(2 * c + r), jnp.float32)
            recip = 1.0 / jnp.maximum(n1, 1.0)
            n0f = (SEQ - n1) * recip
            acc = [zeros, zeros, zeros, zeros]
            for t in range(SEQ):
                slot = r * SEQ + t
                for k in range(4):
                    acc[k] = acc[k] + rows_v[slot, pl.ds(k * L, L)]
            orow = 2 * c + r
            for k in range(4):
                out_v[orow, pl.ds(k * L, L)] = acc[k] * recip - n0f * row0[k]

    def gather(c, rows_v, sem):
        return pltpu.async_copy(table_hbm.at[idx_v.at[c]], rows_v, sem)

    gather(0, rows0_v, sem0).wait()

    def body(i, carry):
        c0 = 2 * i
        gather(c0 + 1, rows1_v, sem1)
        compute_chunk(c0, rows0_v)
        pltpu.make_async_copy(table_hbm.at[idx_v.at[c0 + 1]], rows1_v,
                              sem1).wait()

        @pl.when(c0 + 2 < N_CHUNKS)
        def _():
            gather(c0 + 2, rows0_v, sem0)
        compute_chunk(c0 + 1, rows1_v)

        @pl.when(c0 + 2 < N_CHUNKS)
        def _():
            pltpu.make_async_copy(table_hbm.at[idx_v.at[c0 + 2]], rows0_v,
                                  sem0).wait()
        return carry

    lax.fori_loop(0, N_CHUNKS // 2, body, 0)

    pltpu.sync_copy(out_v, out_hbm.at[pl.ds(wid * B_PER_W, B_PER_W)])


@jax.jit
def kernel(token_ids, table):
    tokens_pad = jnp.pad(token_ids, ((0, 0), (0, SEQ_PAD - SEQ)))
    table_pad = jnp.pad(table, ((0, 0), (0, DIM_PAD - DIM)))
    mesh = plsc.VectorSubcoreMesh(core_axis_name="c", subcore_axis_name="s")
    f = functools.partial(
        pl.kernel,
        mesh=mesh,
        compiler_params=pltpu.CompilerParams(use_tc_tiling_on_sc=True),
        out_type=jax.ShapeDtypeStruct((BATCH, DIM), jnp.float32),
        scratch_types=[
            pltpu.VMEM((B_PER_W, SEQ_PAD), jnp.int32),
            pltpu.VMEM((N_CHUNKS, CHUNK_IDX), jnp.int32),
            pltpu.VMEM((CHUNK_IDX, DIM_PAD), jnp.float32),
            pltpu.VMEM((CHUNK_IDX, DIM_PAD), jnp.float32),
            pltpu.VMEM((1, DIM_PAD), jnp.float32),
            pltpu.VMEM((B_PER_W, DIM), jnp.float32),
            pltpu.SemaphoreType.DMA,
            pltpu.SemaphoreType.DMA,
        ],
    )(_encoder_kernel)
    return f(tokens_pad, table_pad)


# speedup vs baseline: 1.1610x; 1.0386x over previous
import functools

import jax
import jax.numpy as jnp
from jax import lax
from jax.experimental import pallas as pl
from jax.experimental.pallas import tpu as pltpu
from jax.experimental.pallas import tpu_sc as plsc

VOCAB = 1000000
DIM = 64
BATCH = 4096
SEQ = 50
PAD_IDX = 0

L = 16
NW = 32
B_PER_W = BATCH // NW
SEQ_PAD = 128
ROWS_PER_CHUNK = 2
CHUNK_IDX = ROWS_PER_CHUNK * SEQ
N_CHUNKS = B_PER_W // ROWS_PER_CHUNK


DIM_PAD = 128


def _encoder_kernel(tokens_hbm, table_hbm, out_hbm,
                    idxp_v, idx_v, rows0_v, rows1_v, rows2_v, rows3_v,
                    row0_v, out_v, sem0, sem1, sem2, sem3):
    cid = lax.axis_index("c")
    sid = lax.axis_index("s")
    wid = sid * 2 + cid

    pltpu.sync_copy(tokens_hbm.at[pl.ds(wid * B_PER_W, B_PER_W)], idxp_v)
    pltpu.sync_copy(table_hbm.at[pl.ds(0, 1)], row0_v)

    for r in range(B_PER_W):
        c, half = r // 2, (r % 2) * SEQ
        for off in (0, L, 2 * L, 34):
            idx_v[c, pl.ds(half + off, L)] = idxp_v[r, pl.ds(off, L)]

    one = jnp.ones((L,), jnp.int32)
    izero = jnp.zeros((L,), jnp.int32)
    zeros = jnp.zeros((L,), jnp.float32)
    row0 = [row0_v[0, pl.ds(k * L, L)] for k in range(4)]

    def count_nonpad(row):
        cnt = izero
        for off in (0, L, 2 * L, 3 * L):
            toks = idxp_v[row, pl.ds(off, L)]
            cnt = cnt + jnp.where(toks != PAD_IDX, one, izero)
        parts = [cnt[i] for i in range(L)]
        while len(parts) > 1:
            parts = [parts[i] + parts[i + 1] for i in range(0, len(parts), 2)]
        return parts[0]

    def compute_chunk(c, rows_v):
        for r in range(ROWS_PER_CHUNK):
            n1 = jnp.full((L,), count_nonpad(2 * c + r), jnp.float32)
            recip = 1.0 / jnp.maximum(n1, 1.0)
            n0f = (SEQ - n1) * recip
            acc = [zeros, zeros, zeros, zeros]
            for t in range(SEQ):
                slot = r * SEQ + t
                for k in range(4):
                    acc[k] = acc[k] + rows_v[slot, pl.ds(k * L, L)]
            orow = 2 * c + r
            for k in range(4):
                out_v[orow, pl.ds(k * L, L)] = acc[k] * recip - n0f * row0[k]

    bufs = (rows0_v, rows1_v, rows2_v, rows3_v)
    sems = (sem0, sem1, sem2, sem3)
    nbuf = 4

    def gather(c, b):
        return pltpu.async_copy(table_hbm.at[idx_v.at[c]], bufs[b], sems[b])

    def wait(c, b):
        pltpu.make_async_copy(table_hbm.at[idx_v.at[c]], bufs[b],
                              sems[b]).wait()

    for b in range(nbuf - 1):
        gather(b, b)

    def body(i, carry):
        c0 = nbuf * i
        for j in range(nbuf):
            c = c0 + j
            nxt = c + nbuf - 1
            nxt_b = (j + nbuf - 1) % nbuf

            @pl.when(nxt < N_CHUNKS)
            def _():
                gather(nxt, nxt_b)
            wait(c, j)
            compute_chunk(c, bufs[j])
        return carry

    lax.fori_loop(0, N_CHUNKS // nbuf, body, 0)

    pltpu.sync_copy(out_v, out_hbm.at[pl.ds(wid * B_PER_W, B_PER_W)])


@jax.jit
def kernel(token_ids, table):
    tokens_pad = jnp.pad(token_ids, ((0, 0), (0, SEQ_PAD - SEQ)))
    table_pad = jnp.pad(table, ((0, 0), (0, DIM_PAD - DIM)))
    mesh = plsc.VectorSubcoreMesh(core_axis_name="c", subcore_axis_name="s")
    f = functools.partial(
        pl.kernel,
        mesh=mesh,
        compiler_params=pltpu.CompilerParams(use_tc_tiling_on_sc=True),
        out_type=jax.ShapeDtypeStruct((BATCH, DIM), jnp.float32),
        scratch_types=[
            pltpu.VMEM((B_PER_W, SEQ_PAD), jnp.int32),
            pltpu.VMEM((N_CHUNKS, CHUNK_IDX), jnp.int32),
            pltpu.VMEM((CHUNK_IDX, DIM_PAD), jnp.float32),
            pltpu.VMEM((CHUNK_IDX, DIM_PAD), jnp.float32),
            pltpu.VMEM((CHUNK_IDX, DIM_PAD), jnp.float32),
            pltpu.VMEM((CHUNK_IDX, DIM_PAD), jnp.float32),
            pltpu.VMEM((1, DIM_PAD), jnp.float32),
            pltpu.VMEM((B_PER_W, DIM), jnp.float32),
            pltpu.SemaphoreType.DMA,
            pltpu.SemaphoreType.DMA,
            pltpu.SemaphoreType.DMA,
            pltpu.SemaphoreType.DMA,
        ],
    )(_encoder_kernel)
    return f(tokens_pad, table_pad)
